# baseline (speedup 1.0000x reference)
import functools

import jax
import jax.numpy as jnp
from jax import lax
from jax.experimental import pallas as pl
from jax.experimental.pallas import tpu as pltpu
from jax.experimental.pallas import tpu_sc as plsc

B = 1024
D = 64
DP = 128
SEG = 6
ROWS = SEG * B
OUTC = 2 * B
RBLK = 512
H1 = 1024
H2 = 512

_NC = 2
_NS = 16
_NW = _NC * _NS
_RPW = B // _NW
_NBUF = 4

_OFF_YR, _OFF_MO, _OFF_DW, _OFF_HR = 0, 30, 42, 49


def _sc_gather(pid, uid, yr, mo, dw, hr, ptT, utT, smallsT):
    mesh = plsc.VectorSubcoreMesh(
        core_axis_name="c", subcore_axis_name="s",
        num_cores=_NC, num_subcores=_NS)

    @functools.partial(
        pl.kernel,
        out_type=jax.ShapeDtypeStruct((ROWS, DP), jnp.float32),
        mesh=mesh,
        scratch_types=[
            pltpu.VMEM((SEG, _RPW), jnp.int32),
            pltpu.VMEM((_NBUF, D, 128), jnp.float32),
            pltpu.VMEM((D, 128), jnp.float32),
            pltpu.VMEM((_RPW, DP), jnp.float32),
            pltpu.SemaphoreType.DMA,
            pltpu.SemaphoreType.DMA,
        ],
        compiler_params=pltpu.CompilerParams(needs_layout_passes=False),
    )
    def gather_kernel(pid_h, uid_h, yr_h, mo_h, dw_h, hr_h,
                      ptT_h, utT_h, smallsT_h,
                      out_h, idx_v, panel_v, small_v, ebuf_v, psem, ssem):
        wid = lax.axis_index("s") * _NC + lax.axis_index("c")
        base = wid * _RPW
        zeros16 = jnp.zeros((16,), jnp.float32)
        lane16 = lax.iota(jnp.int32, 16)

        pltpu.async_copy(smallsT_h, small_v, ssem).wait()
        idxs = (pid_h, uid_h, yr_h, mo_h, dw_h, hr_h)
        for s in range(SEG):
            pltpu.sync_copy(idxs[s].at[pl.ds(base, _RPW)], idx_v.at[s])
        for j in range(_RPW):
            for h in range(D // 16, DP // 16):
                ebuf_v[j, pl.ds(h * 16, 16)] = zeros16

        def extract(src_ref, c_off, j):
            cols = jnp.full((16,), c_off, jnp.int32)
            for h in range(D // 16):
                vals = plsc.load_gather(src_ref, [lane16 + h * 16, cols])
                ebuf_v[j, pl.ds(h * 16, 16)] = vals

        for s, tbl_h in ((0, ptT_h), (1, utT_h)):
            ids = []
            for g in range(_RPW // 16):
                vec = idx_v[s, pl.ds(g * 16, 16)]
                for t in range(16):
                    ids.append(vec[t])
            cps = [None] * _RPW

            def issue(j):
                c_al = pl.multiple_of((ids[j] // 128) * 128, 128)
                cps[j] = pltpu.async_copy(
                    tbl_h.at[:, pl.ds(c_al, 128)],
                    panel_v.at[j % _NBUF], psem)

            for j in range(_NBUF):
                issue(j)
            for j in range(_RPW):
                cps[j].wait()
                extract(panel_v.at[j % _NBUF], ids[j] % 128, j)
                if j + _NBUF < _RPW:
                    issue(j + _NBUF)
            pltpu.sync_copy(ebuf_v, out_h.at[pl.ds(s * B + base, _RPW), :])

        for s, off in ((2, _OFF_YR), (3, _OFF_MO), (4, _OFF_DW), (5, _OFF_HR)):
            for g in range(_RPW // 16):
                vec = idx_v[s, pl.ds(g * 16, 16)]
                for t in range(16):
                    extract(small_v, vec[t] + off, g * 16 + t)
            pltpu.sync_copy(ebuf_v, out_h.at[pl.ds(s * B + base, _RPW), :])

    return gather_kernel(pid, uid, yr, mo, dw, hr, ptT, utT, smallsT)


def _mlp_body(e_ref, w1_ref, b1_ref, w2_ref, b2_ref, w3_ref,
              pid_ref, uid_ref, wa_ref, wb_ref, c_ref, o_ref):
    h1 = jnp.dot(e_ref[...], w1_ref[...], preferred_element_type=jnp.float32)
    h1 = jnp.maximum(h1 + b1_ref[...], 0.0)
    h2 = jnp.dot(h1, w2_ref[...], preferred_element_type=jnp.float32)
    h2 = jnp.maximum(h2 + b2_ref[...], 0.0)
    d = jnp.dot(h2, w3_ref[...], preferred_element_type=jnp.float32)
    wide = pid_ref[...] * wa_ref[...] + uid_ref[...] * wb_ref[...] + c_ref[...]
    col = lax.broadcasted_iota(jnp.int32, (1, OUTC), 1)
    odd = (col & 1) == 1
    d0 = jnp.broadcast_to(d[:, 0:1], (RBLK, OUTC))
    d1 = jnp.broadcast_to(d[:, 1:2], (RBLK, OUTC))
    dsel = jnp.where(odd, d1, d0)
    o_ref[...] = jax.nn.sigmoid(dsel + wide)


def _mlp_call(E, W1Tp, b1r, W2T, b2r, W3p, pidf, uidf, wav, wbv, cvec):
    nblk = ROWS // RBLK
    full = lambda i: (0, 0)
    return pl.pallas_call(
        _mlp_body,
        grid=(nblk,),
        in_specs=[
            pl.BlockSpec((RBLK, DP), lambda i: (i, 0)),
            pl.BlockSpec((DP, H1), full),
            pl.BlockSpec((1, H1), full),
            pl.BlockSpec((H1, H2), full),
            pl.BlockSpec((1, H2), full),
            pl.BlockSpec((H2, 128), full),
            pl.BlockSpec((1, OUTC), full),
            pl.BlockSpec((1, OUTC), full),
            pl.BlockSpec((1, OUTC), full),
            pl.BlockSpec((1, OUTC), full),
            pl.BlockSpec((1, OUTC), full),
        ],
        out_specs=pl.BlockSpec((RBLK, OUTC), lambda i: (i, 0)),
        out_shape=jax.ShapeDtypeStruct((ROWS, OUTC), jnp.float32),
    )(E, W1Tp, b1r, W2T, b2r, W3p, pidf, uidf, wav, wbv, cvec)


def kernel(product_id, user_id, year, month, day_of_week, hour,
           min_year, max_year,
           product_table, user_table, year_table, month_table,
           day_week_table, time_day_table,
           wide_W, wide_b, W1, b1, W2, b2, W3, b3):
    pid = product_id.reshape(-1).astype(jnp.int32)
    uid = user_id.reshape(-1).astype(jnp.int32)
    yr = year.reshape(-1).astype(jnp.int32)
    mo = month.reshape(-1).astype(jnp.int32)
    dw = day_of_week.reshape(-1).astype(jnp.int32)
    hr = hour.reshape(-1).astype(jnp.int32)

    smallsT = jnp.zeros((D, 128), jnp.float32)
    smallsT = smallsT.at[:, _OFF_YR:_OFF_YR + 30].set(year_table.T)
    smallsT = smallsT.at[:, _OFF_MO:_OFF_MO + 12].set(month_table.T)
    smallsT = smallsT.at[:, _OFF_DW:_OFF_DW + 7].set(day_week_table.T)
    smallsT = smallsT.at[:, _OFF_HR:_OFF_HR + 24].set(time_day_table.T)

    E = _sc_gather(pid, uid, yr, mo, dw, hr,
                   product_table.T, user_table.T, smallsT)

    W1Tp = jnp.zeros((DP, H1), jnp.float32).at[:D, :].set(W1.T)
    b1r = b1.reshape(1, H1)
    W2T = W2.T
    b2r = b2.reshape(1, H2)
    W3p = jnp.zeros((H2, 128), jnp.float32).at[:, :2].set(W3.T)

    pidf = jnp.repeat(pid.astype(jnp.float32), 2).reshape(1, OUTC)
    uidf = jnp.repeat(uid.astype(jnp.float32), 2).reshape(1, OUTC)
    wav = jnp.full((1, OUTC), wide_W[0, 0], jnp.float32)
    wbv = jnp.full((1, OUTC), wide_W[0, 1], jnp.float32)
    cvec = (wide_b[0] + jnp.tile(b3, B)).reshape(1, OUTC)

    out2 = _mlp_call(E, W1Tp, b1r, W2T, b2r, W3p, pidf, uidf, wav, wbv, cvec)
    return out2.reshape(ROWS, B, 2)


# speedup vs baseline: 1.5286x; 1.5286x over previous
import functools

import jax
import jax.numpy as jnp
from jax import lax
from jax.experimental import pallas as pl
from jax.experimental.pallas import tpu as pltpu
from jax.experimental.pallas import tpu_sc as plsc

B = 1024
D = 64
DP = 128
SEG = 6
ROWS = SEG * B
OUTC = 2 * B
RBLK = 512
H1 = 1024
H2 = 512

_NC = 2
_NS = 16
_NW = _NC * _NS
_RPW = B // _NW
_NBUF = 4

_OFF_YR, _OFF_MO, _OFF_DW, _OFF_HR = 0, 30, 42, 49


def _sc_gather(pid, uid, yr, mo, dw, hr, ptT, utT, smallsT):
    mesh = plsc.VectorSubcoreMesh(
        core_axis_name="c", subcore_axis_name="s",
        num_cores=_NC, num_subcores=_NS)

    @functools.partial(
        pl.kernel,
        out_type=jax.ShapeDtypeStruct((ROWS, DP), jnp.float32),
        mesh=mesh,
        scratch_types=[
            pltpu.VMEM((SEG, _RPW), jnp.int32),
            pltpu.VMEM((_NBUF, D, 128), jnp.float32),
            pltpu.VMEM((D, 128), jnp.float32),
            pltpu.VMEM((_RPW, DP), jnp.float32),
            pltpu.SemaphoreType.DMA,
            pltpu.SemaphoreType.DMA,
        ],
        compiler_params=pltpu.CompilerParams(needs_layout_passes=False),
    )
    def gather_kernel(pid_h, uid_h, yr_h, mo_h, dw_h, hr_h,
                      ptT_h, utT_h, smallsT_h,
                      out_h, idx_v, panel_v, small_v, ebuf_v, psem, ssem):
        wid = lax.axis_index("s") * _NC + lax.axis_index("c")
        base = wid * _RPW
        zeros16 = jnp.zeros((16,), jnp.float32)
        lane16 = lax.iota(jnp.int32, 16)

        pltpu.async_copy(smallsT_h, small_v, ssem).wait()
        idxs = (pid_h, uid_h, yr_h, mo_h, dw_h, hr_h)
        for s in range(SEG):
            pltpu.sync_copy(idxs[s].at[pl.ds(base, _RPW)], idx_v.at[s])
        for j in range(_RPW):
            for h in range(D // 16, DP // 16):
                ebuf_v[j, pl.ds(h * 16, 16)] = zeros16

        def extract(src_ref, c_off, j):
            cols = jnp.full((16,), c_off, jnp.int32)
            for h in range(D // 16):
                vals = plsc.load_gather(src_ref, [lane16 + h * 16, cols])
                ebuf_v[j, pl.ds(h * 16, 16)] = vals

        for s, tbl_h in ((0, ptT_h), (1, utT_h)):
            ids = []
            for g in range(_RPW // 16):
                vec = idx_v[s, pl.ds(g * 16, 16)]
                for t in range(16):
                    ids.append(vec[t])
            cps = [None] * _RPW

            def issue(j):
                c_al = pl.multiple_of((ids[j] // 128) * 128, 128)
                cps[j] = pltpu.async_copy(
                    tbl_h.at[:, pl.ds(c_al, 128)],
                    panel_v.at[j % _NBUF], psem)

            for j in range(_NBUF):
                issue(j)
            for j in range(_RPW):
                cps[j].wait()
                extract(panel_v.at[j % _NBUF], ids[j] % 128, j)
                if j + _NBUF < _RPW:
                    issue(j + _NBUF)
            pltpu.sync_copy(ebuf_v, out_h.at[pl.ds(s * B + base, _RPW), :])

        for s, off in ((2, _OFF_YR), (3, _OFF_MO), (4, _OFF_DW), (5, _OFF_HR)):
            for g in range(_RPW // 16):
                vec = idx_v[s, pl.ds(g * 16, 16)]
                for t in range(16):
                    extract(small_v, vec[t] + off, g * 16 + t)
            pltpu.sync_copy(ebuf_v, out_h.at[pl.ds(s * B + base, _RPW), :])

    return gather_kernel(pid, uid, yr, mo, dw, hr, ptT, utT, smallsT)


def _mlp_body(e_ref, w1_ref, b1_ref, w2_ref, b2_ref, w3_ref, b3_ref,
              pid_ref, uid_ref, wa_ref, wb_ref, wc_ref, o_ref):
    h1 = jnp.dot(e_ref[...], w1_ref[...], preferred_element_type=jnp.float32)
    h1 = jnp.maximum(h1 + b1_ref[...], 0.0)
    h2 = lax.dot_general(h1, w2_ref[...], (((1,), (1,)), ((), ())),
                         preferred_element_type=jnp.float32)
    h2 = jnp.maximum(h2 + b2_ref[...], 0.0)
    d8 = lax.dot_general(h2, w3_ref[...], (((1,), (1,)), ((), ())),
                         preferred_element_type=jnp.float32)
    dt = jnp.transpose(d8, (1, 0))[0:2, :] + b3_ref[...]
    x = dt.reshape(2, RBLK // 128, 128).transpose(1, 0, 2)
    x = x.reshape(RBLK // 128 * 2, 128)
    wide = pid_ref[...] * wa_ref[...] + uid_ref[...] * wb_ref[...] + wc_ref[...]
    o_ref[...] = jax.nn.sigmoid(x[None, :, :] + wide)


def _mlp_call(E, W1Tp, b1r, W2, b2r, W3p8, b3c, pid3, uid3, wa3, wb3, wc3):
    nblk = ROWS // RBLK
    full = lambda i: (0, 0)
    full3 = lambda i: (0, 0, 0)
    xb = RBLK // 128 * 2
    return pl.pallas_call(
        _mlp_body,
        grid=(nblk,),
        in_specs=[
            pl.BlockSpec((RBLK, DP), lambda i: (i, 0)),
            pl.BlockSpec((DP, H1), full),
            pl.BlockSpec((1, H1), full),
            pl.BlockSpec((H2, H1), full),
            pl.BlockSpec((1, H2), full),
            pl.BlockSpec((8, H2), full),
            pl.BlockSpec((2, 1), full),
            pl.BlockSpec((B, 1, 1), full3),
            pl.BlockSpec((B, 1, 1), full3),
            pl.BlockSpec((B, 1, 1), full3),
            pl.BlockSpec((B, 1, 1), full3),
            pl.BlockSpec((B, 1, 1), full3),
        ],
        out_specs=pl.BlockSpec((B, xb, 128), lambda i: (0, i, 0)),
        out_shape=jax.ShapeDtypeStruct((B, ROWS // 128 * 2, 128), jnp.float32),
    )(E, W1Tp, b1r, W2, b2r, W3p8, b3c, pid3, uid3, wa3, wb3, wc3)


def kernel(product_id, user_id, year, month, day_of_week, hour,
           min_year, max_year,
           product_table, user_table, year_table, month_table,
           day_week_table, time_day_table,
           wide_W, wide_b, W1, b1, W2, b2, W3, b3):
    pid = product_id.reshape(-1).astype(jnp.int32)
    uid = user_id.reshape(-1).astype(jnp.int32)
    yr = year.reshape(-1).astype(jnp.int32)
    mo = month.reshape(-1).astype(jnp.int32)
    dw = day_of_week.reshape(-1).astype(jnp.int32)
    hr = hour.reshape(-1).astype(jnp.int32)

    smallsT = jnp.zeros((D, 128), jnp.float32)
    smallsT = smallsT.at[:, _OFF_YR:_OFF_YR + 30].set(year_table.T)
    smallsT = smallsT.at[:, _OFF_MO:_OFF_MO + 12].set(month_table.T)
    smallsT = smallsT.at[:, _OFF_DW:_OFF_DW + 7].set(day_week_table.T)
    smallsT = smallsT.at[:, _OFF_HR:_OFF_HR + 24].set(time_day_table.T)

    E = _sc_gather(pid, uid, yr, mo, dw, hr,
                   product_table.T, user_table.T, smallsT)

    W1Tp = jnp.zeros((DP, H1), jnp.float32).at[:D, :].set(W1.T)
    b1r = b1.reshape(1, H1)
    b2r = b2.reshape(1, H2)
    W3p8 = jnp.zeros((8, H2), jnp.float32).at[:2, :].set(W3)
    b3c = b3.reshape(2, 1)

    pid3 = pid.astype(jnp.float32).reshape(B, 1, 1)
    uid3 = uid.astype(jnp.float32).reshape(B, 1, 1)
    wa3 = jnp.full((B, 1, 1), wide_W[0, 0], jnp.float32)
    wb3 = jnp.full((B, 1, 1), wide_W[0, 1], jnp.float32)
    wc3 = jnp.full((B, 1, 1), wide_b[0], jnp.float32)

    out3 = _mlp_call(E, W1Tp, b1r, W2, b2r, W3p8, b3c, pid3, uid3, wa3, wb3, wc3)
    out4 = out3.reshape(B, ROWS // 128, 2, 128).transpose(1, 3, 0, 2)
    return out4.reshape(ROWS, B, 2)


# speedup vs baseline: 3.0510x; 1.9960x over previous
import functools

import jax
import jax.numpy as jnp
from jax import lax
from jax.experimental import pallas as pl
from jax.experimental.pallas import tpu as pltpu
from jax.experimental.pallas import tpu_sc as plsc

B = 1024
D = 64
DP = 128
SEG = 6
ROWS = SEG * B
OUTC = 2 * B
RBLK = 512
H1 = 1024
H2 = 512

_NC = 2
_NS = 16
_NW = _NC * _NS
_RPW = B // _NW
_NBUF = 4

_OFF_YR, _OFF_MO, _OFF_DW, _OFF_HR = 0, 30, 42, 49


def _sc_gather(pid, uid, yr, mo, dw, hr, ptT, utT, smallsT):
    mesh = plsc.VectorSubcoreMesh(
        core_axis_name="c", subcore_axis_name="s",
        num_cores=_NC, num_subcores=_NS)

    @functools.partial(
        pl.kernel,
        out_type=jax.ShapeDtypeStruct((ROWS, DP), jnp.float32),
        mesh=mesh,
        scratch_types=[
            pltpu.VMEM((SEG, _RPW), jnp.int32),
            pltpu.VMEM((_NBUF, D, 128), jnp.float32),
            pltpu.VMEM((D, 128), jnp.float32),
            pltpu.VMEM((_RPW, DP), jnp.float32),
            pltpu.SemaphoreType.DMA,
            pltpu.SemaphoreType.DMA,
        ],
        compiler_params=pltpu.CompilerParams(needs_layout_passes=False),
    )
    def gather_kernel(pid_h, uid_h, yr_h, mo_h, dw_h, hr_h,
                      ptT_h, utT_h, smallsT_h,
                      out_h, idx_v, panel_v, small_v, ebuf_v, psem, ssem):
        wid = lax.axis_index("s") * _NC + lax.axis_index("c")
        base = wid * _RPW
        zeros16 = jnp.zeros((16,), jnp.float32)
        lane16 = lax.iota(jnp.int32, 16)

        pltpu.async_copy(smallsT_h, small_v, ssem).wait()
        idxs = (pid_h, uid_h, yr_h, mo_h, dw_h, hr_h)
        for s in range(SEG):
            pltpu.sync_copy(idxs[s].at[pl.ds(base, _RPW)], idx_v.at[s])
        for j in range(_RPW):
            for h in range(D // 16, DP // 16):
                ebuf_v[j, pl.ds(h * 16, 16)] = zeros16

        def extract(src_ref, c_off, j):
            cols = jnp.full((16,), c_off, jnp.int32)
            for h in range(D // 16):
                vals = plsc.load_gather(src_ref, [lane16 + h * 16, cols])
                ebuf_v[j, pl.ds(h * 16, 16)] = vals

        for s, tbl_h in ((0, ptT_h), (1, utT_h)):
            ids = []
            for g in range(_RPW // 16):
                vec = idx_v[s, pl.ds(g * 16, 16)]
                for t in range(16):
                    ids.append(vec[t])
            cps = [None] * _RPW

            def issue(j):
                c_al = pl.multiple_of((ids[j] // 128) * 128, 128)
                cps[j] = pltpu.async_copy(
                    tbl_h.at[:, pl.ds(c_al, 128)],
                    panel_v.at[j % _NBUF], psem)

            for j in range(_NBUF):
                issue(j)
            for j in range(_RPW):
                cps[j].wait()
                extract(panel_v.at[j % _NBUF], ids[j] % 128, j)
                if j + _NBUF < _RPW:
                    issue(j + _NBUF)
            pltpu.sync_copy(ebuf_v, out_h.at[pl.ds(s * B + base, _RPW), :])

        for s, off in ((2, _OFF_YR), (3, _OFF_MO), (4, _OFF_DW), (5, _OFF_HR)):
            for g in range(_RPW // 16):
                vec = idx_v[s, pl.ds(g * 16, 16)]
                for t in range(16):
                    extract(small_v, vec[t] + off, g * 16 + t)
            pltpu.sync_copy(ebuf_v, out_h.at[pl.ds(s * B + base, _RPW), :])

    return gather_kernel(pid, uid, yr, mo, dw, hr, ptT, utT, smallsT)


def _mlp_body(e_ref, w1_ref, b1_ref, w2_ref, b2_ref, w3x_ref,
              pid_ref, uid_ref, wa_ref, wb_ref, c_ref, o_ref):
    h1 = jnp.dot(e_ref[...], w1_ref[...], preferred_element_type=jnp.float32)
    h1 = jnp.maximum(h1 + b1_ref[...], 0.0)
    h2 = lax.dot_general(h1, w2_ref[...], (((1,), (1,)), ((), ())),
                         preferred_element_type=jnp.float32)
    h2 = jnp.maximum(h2 + b2_ref[...], 0.0)
    dsel = lax.dot_general(h2, w3x_ref[...], (((1,), (1,)), ((), ())),
                           preferred_element_type=jnp.float32)
    wide16 = pid_ref[...] * wa_ref[...] + uid_ref[...] * wb_ref[...] + c_ref[...]
    o_ref[...] = jax.nn.sigmoid(dsel[:, :, None] + wide16[None, :, :])


def _mlp_call(E, W1Tp, b1r, W2, b2r, W3X, pid16, uid16, wa16, wb16, c16):
    nblk = ROWS // RBLK
    full = lambda i: (0, 0)
    return pl.pallas_call(
        _mlp_body,
        grid=(nblk,),
        in_specs=[
            pl.BlockSpec((RBLK, DP), lambda i: (i, 0)),
            pl.BlockSpec((DP, H1), full),
            pl.BlockSpec((1, H1), full),
            pl.BlockSpec((H2, H1), full),
            pl.BlockSpec((1, H2), full),
            pl.BlockSpec((16, H2), full),
            pl.BlockSpec((16, 128), full),
            pl.BlockSpec((16, 128), full),
            pl.BlockSpec((16, 1), full),
            pl.BlockSpec((16, 1), full),
            pl.BlockSpec((16, 1), full),
        ],
        out_specs=pl.BlockSpec((RBLK, 16, 128), lambda i: (i, 0, 0)),
        out_shape=jax.ShapeDtypeStruct((ROWS, 16, 128), jnp.float32),
    )(E, W1Tp, b1r, W2, b2r, W3X, pid16, uid16, wa16, wb16, c16)


def kernel(product_id, user_id, year, month, day_of_week, hour,
           min_year, max_year,
           product_table, user_table, year_table, month_table,
           day_week_table, time_day_table,
           wide_W, wide_b, W1, b1, W2, b2, W3, b3):
    pid = product_id.reshape(-1).astype(jnp.int32)
    uid = user_id.reshape(-1).astype(jnp.int32)
    yr = year.reshape(-1).astype(jnp.int32)
    mo = month.reshape(-1).astype(jnp.int32)
    dw = day_of_week.reshape(-1).astype(jnp.int32)
    hr = hour.reshape(-1).astype(jnp.int32)

    smallsT = jnp.zeros((D, 128), jnp.float32)
    smallsT = smallsT.at[:, _OFF_YR:_OFF_YR + 30].set(year_table.T)
    smallsT = smallsT.at[:, _OFF_MO:_OFF_MO + 12].set(month_table.T)
    smallsT = smallsT.at[:, _OFF_DW:_OFF_DW + 7].set(day_week_table.T)
    smallsT = smallsT.at[:, _OFF_HR:_OFF_HR + 24].set(time_day_table.T)

    E = _sc_gather(pid, uid, yr, mo, dw, hr,
                   product_table.T, user_table.T, smallsT)

    W1Tp = jnp.zeros((DP, H1), jnp.float32).at[:D, :].set(W1.T)
    b1r = b1.reshape(1, H1)
    b2r = b2.reshape(1, H2)
    W3X = jnp.tile(W3, (8, 1))

    pid16 = jnp.repeat(pid.astype(jnp.float32).reshape(8, 128), 2, axis=0)
    uid16 = jnp.repeat(uid.astype(jnp.float32).reshape(8, 128), 2, axis=0)
    wa16 = jnp.full((16, 1), wide_W[0, 0], jnp.float32)
    wb16 = jnp.full((16, 1), wide_W[0, 1], jnp.float32)
    c16 = (wide_b[0] + jnp.tile(b3, 8)).reshape(16, 1)

    out3 = _mlp_call(E, W1Tp, b1r, W2, b2r, W3X, pid16, uid16, wa16, wb16, c16)
    out4 = out3.reshape(ROWS, 8, 2, 128).transpose(0, 1, 3, 2)
    return out4.reshape(ROWS, B, 2)


# speedup vs baseline: 3.0690x; 1.0059x over previous
import functools

import jax
import jax.numpy as jnp
from jax import lax
from jax.experimental import pallas as pl
from jax.experimental.pallas import tpu as pltpu
from jax.experimental.pallas import tpu_sc as plsc

B = 1024
D = 64
DP = 128
SEG = 6
ROWS = SEG * B
OUTC = 2 * B
RBLK = 512
H1 = 1024
H2 = 512

_NC = 2
_NS = 16
_NW = _NC * _NS
_RPW = B // _NW
_NBUF = 8

_OFF_YR, _OFF_MO, _OFF_DW, _OFF_HR = 0, 30, 42, 49


def _sc_gather(pid, uid, yr, mo, dw, hr, ptT, utT, smallsT):
    mesh = plsc.VectorSubcoreMesh(
        core_axis_name="c", subcore_axis_name="s",
        num_cores=_NC, num_subcores=_NS)

    @functools.partial(
        pl.kernel,
        out_type=jax.ShapeDtypeStruct((ROWS, DP), jnp.float32),
        mesh=mesh,
        scratch_types=[
            pltpu.VMEM((SEG, _RPW), jnp.int32),
            pltpu.VMEM((_NBUF, D, 128), jnp.float32),
            pltpu.VMEM((D, 128), jnp.float32),
            pltpu.VMEM((2, _RPW, DP), jnp.float32),
            pltpu.SemaphoreType.DMA,
            pltpu.SemaphoreType.DMA,
        ],
        compiler_params=pltpu.CompilerParams(needs_layout_passes=False),
    )
    def gather_kernel(pid_h, uid_h, yr_h, mo_h, dw_h, hr_h,
                      ptT_h, utT_h, smallsT_h,
                      out_h, idx_v, panel_v, small_v, ebuf_v, psem, ssem):
        wid = lax.axis_index("s") * _NC + lax.axis_index("c")
        base = wid * _RPW
        zeros16 = jnp.zeros((16,), jnp.float32)
        lane16 = lax.iota(jnp.int32, 16)

        pltpu.async_copy(smallsT_h, small_v, ssem).wait()
        idxs = (pid_h, uid_h, yr_h, mo_h, dw_h, hr_h)
        for s in range(SEG):
            pltpu.sync_copy(idxs[s].at[pl.ds(base, _RPW)], idx_v.at[s])
        for sb in range(2):
            for j in range(_RPW):
                for h in range(D // 16, DP // 16):
                    ebuf_v[sb, j, pl.ds(h * 16, 16)] = zeros16

        def extract(src_ref, c_off, sb, j):
            cols = jnp.full((16,), c_off, jnp.int32)
            for h in range(D // 16):
                vals = plsc.load_gather(src_ref, [lane16 + h * 16, cols])
                ebuf_v[sb, j, pl.ds(h * 16, 16)] = vals

        ids = []
        for s, tbl_h in ((0, ptT_h), (1, utT_h)):
            for g in range(_RPW // 16):
                vec = idx_v[s, pl.ds(g * 16, 16)]
                for t in range(16):
                    ids.append((s, g * 16 + t, vec[t], tbl_h))
        nbig = len(ids)
        cps = [None] * nbig

        def issue(n):
            _, _, c, tbl_h = ids[n]
            c_al = pl.multiple_of((c // 128) * 128, 128)
            cps[n] = pltpu.async_copy(
                tbl_h.at[:, pl.ds(c_al, 128)],
                panel_v.at[n % _NBUF], psem)

        for n in range(_NBUF):
            issue(n)
        for n in range(nbig):
            s, j, c, _ = ids[n]
            cps[n].wait()
            extract(panel_v.at[n % _NBUF], c % 128, s, j)
            if n + _NBUF < nbig:
                issue(n + _NBUF)
        for s in range(2):
            pltpu.sync_copy(ebuf_v.at[s],
                            out_h.at[pl.ds(s * B + base, _RPW), :])

        for s, off in ((2, _OFF_YR), (3, _OFF_MO), (4, _OFF_DW), (5, _OFF_HR)):
            for g in range(_RPW // 16):
                vec = idx_v[s, pl.ds(g * 16, 16)]
                for t in range(16):
                    extract(small_v, vec[t] + off, 0, g * 16 + t)
            pltpu.sync_copy(ebuf_v.at[0],
                            out_h.at[pl.ds(s * B + base, _RPW), :])

    return gather_kernel(pid, uid, yr, mo, dw, hr, ptT, utT, smallsT)


def _mlp_body(e_ref, w1_ref, b1_ref, w2_ref, b2_ref, w3x_ref,
              pid_ref, uid_ref, wa_ref, wb_ref, c_ref, o_ref):
    h1 = jnp.dot(e_ref[...], w1_ref[...], preferred_element_type=jnp.float32)
    h1 = jnp.maximum(h1 + b1_ref[...], 0.0)
    h2 = lax.dot_general(h1, w2_ref[...], (((1,), (1,)), ((), ())),
                         preferred_element_type=jnp.float32)
    h2 = jnp.maximum(h2 + b2_ref[...], 0.0)
    dsel = lax.dot_general(h2, w3x_ref[...], (((1,), (1,)), ((), ())),
                           preferred_element_type=jnp.float32)
    wide16 = pid_ref[...] * wa_ref[...] + uid_ref[...] * wb_ref[...] + c_ref[...]
    o_ref[...] = jax.nn.sigmoid(dsel[:, :, None] + wide16[None, :, :])


def _mlp_call(E, W1Tp, b1r, W2, b2r, W3X, pid16, uid16, wa16, wb16, c16):
    nblk = ROWS // RBLK
    full = lambda i: (0, 0)
    return pl.pallas_call(
        _mlp_body,
        grid=(nblk,),
        in_specs=[
            pl.BlockSpec((RBLK, DP), lambda i: (i, 0)),
            pl.BlockSpec((DP, H1), full),
            pl.BlockSpec((1, H1), full),
            pl.BlockSpec((H2, H1), full),
            pl.BlockSpec((1, H2), full),
            pl.BlockSpec((16, H2), full),
            pl.BlockSpec((16, 128), full),
            pl.BlockSpec((16, 128), full),
            pl.BlockSpec((16, 1), full),
            pl.BlockSpec((16, 1), full),
            pl.BlockSpec((16, 1), full),
        ],
        out_specs=pl.BlockSpec((RBLK, 16, 128), lambda i: (i, 0, 0)),
        out_shape=jax.ShapeDtypeStruct((ROWS, 16, 128), jnp.float32),
    )(E, W1Tp, b1r, W2, b2r, W3X, pid16, uid16, wa16, wb16, c16)


def kernel(product_id, user_id, year, month, day_of_week, hour,
           min_year, max_year,
           product_table, user_table, year_table, month_table,
           day_week_table, time_day_table,
           wide_W, wide_b, W1, b1, W2, b2, W3, b3):
    pid = product_id.reshape(-1).astype(jnp.int32)
    uid = user_id.reshape(-1).astype(jnp.int32)
    yr = year.reshape(-1).astype(jnp.int32)
    mo = month.reshape(-1).astype(jnp.int32)
    dw = day_of_week.reshape(-1).astype(jnp.int32)
    hr = hour.reshape(-1).astype(jnp.int32)

    smallsT = jnp.zeros((D, 128), jnp.float32)
    smallsT = smallsT.at[:, _OFF_YR:_OFF_YR + 30].set(year_table.T)
    smallsT = smallsT.at[:, _OFF_MO:_OFF_MO + 12].set(month_table.T)
    smallsT = smallsT.at[:, _OFF_DW:_OFF_DW + 7].set(day_week_table.T)
    smallsT = smallsT.at[:, _OFF_HR:_OFF_HR + 24].set(time_day_table.T)

    E = _sc_gather(pid, uid, yr, mo, dw, hr,
                   product_table.T, user_table.T, smallsT)

    W1Tp = jnp.zeros((DP, H1), jnp.float32).at[:D, :].set(W1.T)
    b1r = b1.reshape(1, H1)
    b2r = b2.reshape(1, H2)
    W3X = jnp.tile(W3, (8, 1))

    pid16 = jnp.repeat(pid.astype(jnp.float32).reshape(8, 128), 2, axis=0)
    uid16 = jnp.repeat(uid.astype(jnp.float32).reshape(8, 128), 2, axis=0)
    wa16 = jnp.full((16, 1), wide_W[0, 0], jnp.float32)
    wb16 = jnp.full((16, 1), wide_W[0, 1], jnp.float32)
    c16 = (wide_b[0] + jnp.tile(b3, 8)).reshape(16, 1)

    out3 = _mlp_call(E, W1Tp, b1r, W2, b2r, W3X, pid16, uid16, wa16, wb16, c16)
    out4 = out3.reshape(ROWS, 8, 2, 128).transpose(0, 1, 3, 2)
    return out4.reshape(ROWS, B, 2)


# speedup vs baseline: 3.1038x; 1.0113x over previous
import functools

import jax
import jax.numpy as jnp
from jax import lax
from jax.experimental import pallas as pl
from jax.experimental.pallas import tpu as pltpu
from jax.experimental.pallas import tpu_sc as plsc

B = 1024
D = 64
DP = 128
SEG = 6
ROWS = SEG * B
OUTC = 2 * B
RBLK = 512
H1 = 1024
H2 = 512

_NC = 2
_NS = 16
_NW = _NC * _NS
_RPW = B // _NW
_NBUF = 8

_OFF_YR, _OFF_MO, _OFF_DW, _OFF_HR = 0, 30, 42, 49


def _sc_gather(ids_all, ptT, utT, smallsR):
    mesh = plsc.VectorSubcoreMesh(
        core_axis_name="c", subcore_axis_name="s",
        num_cores=_NC, num_subcores=_NS)

    @functools.partial(
        pl.kernel,
        out_type=jax.ShapeDtypeStruct((ROWS, DP), jnp.float32),
        mesh=mesh,
        scratch_types=[
            pltpu.VMEM((SEG, _RPW), jnp.int32),
            pltpu.VMEM((_NBUF, D, 128), jnp.float32),
            pltpu.VMEM((128, D), jnp.float32),
            pltpu.VMEM((2, _RPW, DP), jnp.float32),
            pltpu.SemaphoreType.DMA,
            pltpu.SemaphoreType.DMA,
        ],
        compiler_params=pltpu.CompilerParams(needs_layout_passes=False),
    )
    def gather_kernel(ids_h, ptT_h, utT_h, smallsR_h,
                      out_h, idx_v, panel_v, small_v, ebuf_v, psem, ssem):
        wid = lax.axis_index("s") * _NC + lax.axis_index("c")
        base = wid * _RPW
        zeros16 = jnp.zeros((16,), jnp.float32)
        lane16 = lax.iota(jnp.int32, 16)

        pltpu.async_copy(smallsR_h, small_v, ssem).wait()
        for s in range(SEG):
            pltpu.sync_copy(ids_h.at[pl.ds(s * B + base, _RPW)], idx_v.at[s])
        for sb in range(2):
            for j in range(_RPW):
                for h in range(D // 16, DP // 16):
                    ebuf_v[sb, j, pl.ds(h * 16, 16)] = zeros16

        def extract(src_ref, c_off, sb, j):
            cols = jnp.full((16,), c_off, jnp.int32)
            for h in range(D // 16):
                vals = plsc.load_gather(src_ref, [lane16 + h * 16, cols])
                ebuf_v[sb, j, pl.ds(h * 16, 16)] = vals

        ids = []
        for s, tbl_h in ((0, ptT_h), (1, utT_h)):
            for g in range(_RPW // 16):
                vec = idx_v[s, pl.ds(g * 16, 16)]
                for t in range(16):
                    ids.append((s, g * 16 + t, vec[t], tbl_h))
        nbig = len(ids)
        cps = [None] * nbig

        def issue(n):
            _, _, c, tbl_h = ids[n]
            c_al = pl.multiple_of((c // 128) * 128, 128)
            cps[n] = pltpu.async_copy(
                tbl_h.at[:, pl.ds(c_al, 128)],
                panel_v.at[n % _NBUF], psem)

        for n in range(_NBUF):
            issue(n)
        for n in range(nbig):
            s, j, c, _ = ids[n]
            cps[n].wait()
            extract(panel_v.at[n % _NBUF], c % 128, s, j)
            if n + _NBUF < nbig:
                issue(n + _NBUF)
        for s in range(2):
            pltpu.sync_copy(ebuf_v.at[s],
                            out_h.at[pl.ds(s * B + base, _RPW), :])

        for s, off in ((2, _OFF_YR), (3, _OFF_MO), (4, _OFF_DW), (5, _OFF_HR)):
            for g in range(_RPW // 16):
                vec = idx_v[s, pl.ds(g * 16, 16)]
                for t in range(16):
                    rows = jnp.full((16,), vec[t] + off, jnp.int32)
                    for h in range(D // 16):
                        vals = plsc.load_gather(small_v, [rows, lane16 + h * 16])
                        ebuf_v[0, g * 16 + t, pl.ds(h * 16, 16)] = vals
            pltpu.sync_copy(ebuf_v.at[0],
                            out_h.at[pl.ds(s * B + base, _RPW), :])

    return gather_kernel(ids_all, ptT, utT, smallsR)


def _mlp_body(e_ref, w1_ref, b1_ref, w2_ref, b2_ref, w3x_ref,
              pid_ref, uid_ref, wa_ref, wb_ref, c_ref, o_ref):
    h1 = jnp.dot(e_ref[...], w1_ref[...], preferred_element_type=jnp.float32)
    h1 = jnp.maximum(h1 + b1_ref[...], 0.0)
    h2 = lax.dot_general(h1, w2_ref[...], (((1,), (1,)), ((), ())),
                         preferred_element_type=jnp.float32)
    h2 = jnp.maximum(h2 + b2_ref[...], 0.0)
    dsel = lax.dot_general(h2, w3x_ref[...], (((1,), (1,)), ((), ())),
                           preferred_element_type=jnp.float32)
    wide16 = pid_ref[...] * wa_ref[...] + uid_ref[...] * wb_ref[...] + c_ref[...]
    o_ref[...] = jax.nn.sigmoid(dsel[:, :, None] + wide16[None, :, :])


def _mlp_call(E, W1Tp, b1r, W2, b2r, W3X, pid16, uid16, wa16, wb16, c16):
    nblk = ROWS // RBLK
    full = lambda i: (0, 0)
    return pl.pallas_call(
        _mlp_body,
        grid=(nblk,),
        in_specs=[
            pl.BlockSpec((RBLK, DP), lambda i: (i, 0)),
            pl.BlockSpec((DP, H1), full),
            pl.BlockSpec((1, H1), full),
            pl.BlockSpec((H2, H1), full),
            pl.BlockSpec((1, H2), full),
            pl.BlockSpec((16, H2), full),
            pl.BlockSpec((16, 128), full),
            pl.BlockSpec((16, 128), full),
            pl.BlockSpec((16, 1), full),
            pl.BlockSpec((16, 1), full),
            pl.BlockSpec((16, 1), full),
        ],
        out_specs=pl.BlockSpec((RBLK, 16, 128), lambda i: (i, 0, 0)),
        out_shape=jax.ShapeDtypeStruct((ROWS, 16, 128), jnp.float32),
    )(E, W1Tp, b1r, W2, b2r, W3X, pid16, uid16, wa16, wb16, c16)


def kernel(product_id, user_id, year, month, day_of_week, hour,
           min_year, max_year,
           product_table, user_table, year_table, month_table,
           day_week_table, time_day_table,
           wide_W, wide_b, W1, b1, W2, b2, W3, b3):
    ids_all = jnp.concatenate(
        [product_id, user_id, year, month, day_of_week, hour],
        axis=0).reshape(-1).astype(jnp.int32)

    smallsR = jnp.concatenate(
        [year_table, month_table, day_week_table, time_day_table,
         jnp.zeros((128 - 73, D), jnp.float32)], axis=0)

    E = _sc_gather(ids_all, product_table.T, user_table.T, smallsR)

    W1Tp = jnp.zeros((DP, H1), jnp.float32).at[:D, :].set(W1.T)
    b1r = b1.reshape(1, H1)
    b2r = b2.reshape(1, H2)
    W3X = jnp.tile(W3, (8, 1))

    pid16 = jnp.repeat(product_id.astype(jnp.float32).reshape(8, 128), 2, axis=0)
    uid16 = jnp.repeat(user_id.astype(jnp.float32).reshape(8, 128), 2, axis=0)
    wa16 = jnp.full((16, 1), wide_W[0, 0], jnp.float32)
    wb16 = jnp.full((16, 1), wide_W[0, 1], jnp.float32)
    c16 = (wide_b[0] + jnp.tile(b3, 8)).reshape(16, 1)

    out3 = _mlp_call(E, W1Tp, b1r, W2, b2r, W3X, pid16, uid16, wa16, wb16, c16)
    out4 = out3.reshape(ROWS, 8, 2, 128).transpose(0, 1, 3, 2)
    return out4.reshape(ROWS, B, 2)


# speedup vs baseline: 3.3333x; 1.0739x over previous
import functools

import jax
import jax.numpy as jnp
from jax import lax
from jax.experimental import pallas as pl
from jax.experimental.pallas import tpu as pltpu
from jax.experimental.pallas import tpu_sc as plsc

B = 1024
D = 64
DP = 128
SEG = 6
ROWS = SEG * B
OUTC = 2 * B
RBLK = 512
H1 = 1024
H2 = 512

_NC = 2
_NS = 16
_NW = _NC * _NS
_RPW = B // _NW
_NBUF = 8

_OFF_YR, _OFF_MO, _OFF_DW, _OFF_HR = 0, 30, 42, 49


def _sc_gather(ids_all, ptT, utT, yt, mt, dwt, tdt):
    mesh = plsc.VectorSubcoreMesh(
        core_axis_name="c", subcore_axis_name="s",
        num_cores=_NC, num_subcores=_NS)

    @functools.partial(
        pl.kernel,
        out_type=jax.ShapeDtypeStruct((ROWS, DP), jnp.float32),
        mesh=mesh,
        scratch_types=[
            pltpu.VMEM((SEG, _RPW), jnp.int32),
            pltpu.VMEM((_NBUF, D, 128), jnp.float32),
            pltpu.VMEM((128, D), jnp.float32),
            pltpu.VMEM((2, _RPW, DP), jnp.float32),
            pltpu.SemaphoreType.DMA,
            pltpu.SemaphoreType.DMA,
        ],
        compiler_params=pltpu.CompilerParams(needs_layout_passes=False),
    )
    def gather_kernel(ids_h, ptT_h, utT_h, yt_h, mt_h, dwt_h, tdt_h,
                      out_h, idx_v, panel_v, small_v, ebuf_v, psem, ssem):
        wid = lax.axis_index("s") * _NC + lax.axis_index("c")
        base = wid * _RPW
        zeros16 = jnp.zeros((16,), jnp.float32)
        lane16 = lax.iota(jnp.int32, 16)

        scp = [pltpu.async_copy(yt_h, small_v.at[pl.ds(_OFF_YR, 30)], ssem),
               pltpu.async_copy(mt_h, small_v.at[pl.ds(_OFF_MO, 12)], ssem),
               pltpu.async_copy(dwt_h, small_v.at[pl.ds(_OFF_DW, 7)], ssem),
               pltpu.async_copy(tdt_h, small_v.at[pl.ds(_OFF_HR, 24)], ssem)]
        for cp in scp:
            cp.wait()
        for s in range(SEG):
            pltpu.sync_copy(ids_h.at[pl.ds(s * B + base, _RPW)], idx_v.at[s])
        for sb in range(2):
            for j in range(_RPW):
                for h in range(D // 16, DP // 16):
                    ebuf_v[sb, j, pl.ds(h * 16, 16)] = zeros16

        def extract(src_ref, c_off, sb, j):
            cols = jnp.full((16,), c_off, jnp.int32)
            for h in range(D // 16):
                vals = plsc.load_gather(src_ref, [lane16 + h * 16, cols])
                ebuf_v[sb, j, pl.ds(h * 16, 16)] = vals

        ids = []
        for s, tbl_h in ((0, ptT_h), (1, utT_h)):
            for g in range(_RPW // 16):
                vec = idx_v[s, pl.ds(g * 16, 16)]
                for t in range(16):
                    ids.append((s, g * 16 + t, vec[t], tbl_h))
        nbig = len(ids)
        cps = [None] * nbig

        def issue(n):
            _, _, c, tbl_h = ids[n]
            c_al = pl.multiple_of((c // 128) * 128, 128)
            cps[n] = pltpu.async_copy(
                tbl_h.at[:, pl.ds(c_al, 128)],
                panel_v.at[n % _NBUF], psem)

        for n in range(_NBUF):
            issue(n)
        for n in range(nbig):
            s, j, c, _ = ids[n]
            cps[n].wait()
            extract(panel_v.at[n % _NBUF], c % 128, s, j)
            if n + _NBUF < nbig:
                issue(n + _NBUF)
        for s in range(2):
            pltpu.sync_copy(ebuf_v.at[s],
                            out_h.at[pl.ds(s * B + base, _RPW), :])

        for s, off in ((2, _OFF_YR), (3, _OFF_MO), (4, _OFF_DW), (5, _OFF_HR)):
            for g in range(_RPW // 16):
                vec = idx_v[s, pl.ds(g * 16, 16)]
                for t in range(16):
                    rows = jnp.full((16,), vec[t] + off, jnp.int32)
                    for h in range(D // 16):
                        vals = plsc.load_gather(small_v, [rows, lane16 + h * 16])
                        ebuf_v[0, g * 16 + t, pl.ds(h * 16, 16)] = vals
            pltpu.sync_copy(ebuf_v.at[0],
                            out_h.at[pl.ds(s * B + base, _RPW), :])

    return gather_kernel(ids_all, ptT, utT, yt, mt, dwt, tdt)


def _mlp_body(e_ref, w1_ref, b1_ref, w2_ref, b2_ref, w3x_ref,
              pid_ref, uid_ref, wa_ref, wb_ref, c_ref, o_ref):
    h1 = jnp.dot(e_ref[...], w1_ref[...], preferred_element_type=jnp.float32)
    h1 = jnp.maximum(h1 + b1_ref[...], 0.0)
    h2 = lax.dot_general(h1, w2_ref[...], (((1,), (1,)), ((), ())),
                         preferred_element_type=jnp.float32)
    h2 = jnp.maximum(h2 + b2_ref[...], 0.0)
    dsel = lax.dot_general(h2, w3x_ref[...], (((1,), (1,)), ((), ())),
                           preferred_element_type=jnp.float32)
    wide16 = pid_ref[...] * wa_ref[...] + uid_ref[...] * wb_ref[...] + c_ref[...]
    o_ref[...] = jax.nn.sigmoid(dsel[:, :, None] + wide16[None, :, :])


def _mlp_call(E, W1Tp, b1r, W2, b2r, W3X, pid16, uid16, wa16, wb16, c16):
    nblk = ROWS // RBLK
    full = lambda i: (0, 0)
    return pl.pallas_call(
        _mlp_body,
        grid=(nblk,),
        in_specs=[
            pl.BlockSpec((RBLK, DP), lambda i: (i, 0)),
            pl.BlockSpec((DP, H1), full),
            pl.BlockSpec((1, H1), full),
            pl.BlockSpec((H2, H1), full),
            pl.BlockSpec((1, H2), full),
            pl.BlockSpec((16, H2), full),
            pl.BlockSpec((16, 128), full),
            pl.BlockSpec((16, 128), full),
            pl.BlockSpec((16, 1), full),
            pl.BlockSpec((16, 1), full),
            pl.BlockSpec((16, 1), full),
        ],
        out_specs=pl.BlockSpec((RBLK, 16, 128), lambda i: (i, 0, 0)),
        out_shape=jax.ShapeDtypeStruct((ROWS, 16, 128), jnp.float32),
    )(E, W1Tp, b1r, W2, b2r, W3X, pid16, uid16, wa16, wb16, c16)


def kernel(product_id, user_id, year, month, day_of_week, hour,
           min_year, max_year,
           product_table, user_table, year_table, month_table,
           day_week_table, time_day_table,
           wide_W, wide_b, W1, b1, W2, b2, W3, b3):
    ids_all = jnp.concatenate(
        [product_id.T, user_id.T, year.T, month.T, day_of_week.T, hour.T],
        axis=0).reshape(-1).astype(jnp.int32)

    E = _sc_gather(ids_all, product_table.T, user_table.T,
                   year_table, month_table, day_week_table, time_day_table)

    W1Tp = jnp.zeros((DP, H1), jnp.float32).at[:D, :].set(W1.T)
    b1r = b1.reshape(1, H1)
    b2r = b2.reshape(1, H2)
    W3X = jnp.tile(W3, (8, 1))

    pid16 = jnp.repeat(product_id.astype(jnp.float32).reshape(8, 128), 2, axis=0)
    uid16 = jnp.repeat(user_id.astype(jnp.float32).reshape(8, 128), 2, axis=0)
    wa16 = jnp.full((16, 1), wide_W[0, 0], jnp.float32)
    wb16 = jnp.full((16, 1), wide_W[0, 1], jnp.float32)
    c16 = (wide_b[0] + jnp.tile(b3, 8)).reshape(16, 1)

    out3 = _mlp_call(E, W1Tp, b1r, W2, b2r, W3X, pid16, uid16, wa16, wb16, c16)
    out4 = out3.reshape(ROWS, 8, 2, 128).transpose(0, 1, 3, 2)
    return out4.reshape(ROWS, B, 2)


# speedup vs baseline: 3.4665x; 1.0400x over previous
import functools

import jax
import jax.numpy as jnp
from jax import lax
from jax.experimental import pallas as pl
from jax.experimental.pallas import tpu as pltpu
from jax.experimental.pallas import tpu_sc as plsc

B = 1024
D = 64
DP = 128
SEG = 6
ROWS = SEG * B
OUTC = 2 * B
RBLK = 512
H1 = 1024
H2 = 512

_NC = 2
_NS = 16
_NW = _NC * _NS
_RPW = B // _NW
_NBUF = 8

_OFF_YR, _OFF_MO, _OFF_DW, _OFF_HR = 0, 30, 42, 49


def _sc_gather(ids_all, ptT, utT, yt, mt, dwt, tdt):
    mesh = plsc.VectorSubcoreMesh(
        core_axis_name="c", subcore_axis_name="s",
        num_cores=_NC, num_subcores=_NS)

    @functools.partial(
        pl.kernel,
        out_type=jax.ShapeDtypeStruct((ROWS, DP), jnp.float32),
        mesh=mesh,
        scratch_types=[
            pltpu.VMEM((SEG, _RPW), jnp.int32),
            pltpu.VMEM((_NBUF, D, 128), jnp.float32),
            pltpu.VMEM((128, D), jnp.float32),
            pltpu.VMEM((2, _RPW, DP), jnp.float32),
            pltpu.SemaphoreType.DMA,
            pltpu.SemaphoreType.DMA,
        ],
        compiler_params=pltpu.CompilerParams(needs_layout_passes=False),
    )
    def gather_kernel(ids_h, ptT_h, utT_h, yt_h, mt_h, dwt_h, tdt_h,
                      out_h, idx_v, panel_v, small_v, ebuf_v, psem, ssem):
        wid = lax.axis_index("s") * _NC + lax.axis_index("c")
        base = wid * _RPW
        zeros16 = jnp.zeros((16,), jnp.float32)
        lane16 = lax.iota(jnp.int32, 16)

        scp = [pltpu.async_copy(yt_h, small_v.at[pl.ds(_OFF_YR, 30)], ssem),
               pltpu.async_copy(mt_h, small_v.at[pl.ds(_OFF_MO, 12)], ssem),
               pltpu.async_copy(dwt_h, small_v.at[pl.ds(_OFF_DW, 7)], ssem),
               pltpu.async_copy(tdt_h, small_v.at[pl.ds(_OFF_HR, 24)], ssem)]
        for cp in scp:
            cp.wait()
        for s in range(SEG):
            pltpu.sync_copy(ids_h.at[pl.ds(s * B + base, _RPW)], idx_v.at[s])
        for sb in range(2):
            for j in range(_RPW):
                for h in range(D // 16, DP // 16):
                    ebuf_v[sb, j, pl.ds(h * 16, 16)] = zeros16

        def extract(src_ref, c_off, sb, j):
            cols = jnp.full((16,), c_off, jnp.int32)
            for h in range(D // 16):
                vals = plsc.load_gather(src_ref, [lane16 + h * 16, cols])
                ebuf_v[sb, j, pl.ds(h * 16, 16)] = vals

        ids = []
        for s, tbl_h in ((0, ptT_h), (1, utT_h)):
            for g in range(_RPW // 16):
                vec = idx_v[s, pl.ds(g * 16, 16)]
                for t in range(16):
                    ids.append((s, g * 16 + t, vec[t], tbl_h))
        nbig = len(ids)
        cps = [None] * nbig

        def issue(n):
            _, _, c, tbl_h = ids[n]
            c_al = pl.multiple_of((c // 128) * 128, 128)
            cps[n] = pltpu.async_copy(
                tbl_h.at[:, pl.ds(c_al, 128)],
                panel_v.at[n % _NBUF], psem)

        for n in range(_NBUF):
            issue(n)
        for n in range(nbig):
            s, j, c, _ = ids[n]
            cps[n].wait()
            extract(panel_v.at[n % _NBUF], c % 128, s, j)
            if n + _NBUF < nbig:
                issue(n + _NBUF)
        for s in range(2):
            pltpu.sync_copy(ebuf_v.at[s],
                            out_h.at[pl.ds(s * B + base, _RPW), :])

        for s, off in ((2, _OFF_YR), (3, _OFF_MO), (4, _OFF_DW), (5, _OFF_HR)):
            for g in range(_RPW // 16):
                vec = idx_v[s, pl.ds(g * 16, 16)]
                for t in range(16):
                    rows = jnp.full((16,), vec[t] + off, jnp.int32)
                    for h in range(D // 16):
                        vals = plsc.load_gather(small_v, [rows, lane16 + h * 16])
                        ebuf_v[0, g * 16 + t, pl.ds(h * 16, 16)] = vals
            pltpu.sync_copy(ebuf_v.at[0],
                            out_h.at[pl.ds(s * B + base, _RPW), :])

    return gather_kernel(ids_all, ptT, utT, yt, mt, dwt, tdt)


def _mlp_body(e_ref, w1_ref, b1_ref, w2_ref, b2_ref, w3x_ref,
              pu_ref, ww_ref, c_ref, o_ref):
    h1 = jnp.dot(e_ref[...], w1_ref[...], preferred_element_type=jnp.float32)
    h1 = jnp.maximum(h1 + b1_ref[...], 0.0)
    h2 = lax.dot_general(h1, w2_ref[...], (((1,), (1,)), ((), ())),
                         preferred_element_type=jnp.float32)
    h2 = jnp.maximum(h2 + b2_ref[...], 0.0)
    dsel = lax.dot_general(h2, w3x_ref[...], (((1,), (1,)), ((), ())),
                           preferred_element_type=jnp.float32)
    wide16 = (pu_ref[0] * ww_ref[0, 0] + pu_ref[1] * ww_ref[0, 1]
              + c_ref[...])
    o_ref[...] = jax.nn.sigmoid(dsel[:, :, None] + wide16[None, :, :])


def _mlp_call(E, W1Tp, b1r, W2, b2r, W3X, pu16, wide_W, c16):
    nblk = ROWS // RBLK
    full = lambda i: (0, 0)
    return pl.pallas_call(
        _mlp_body,
        grid=(nblk,),
        in_specs=[
            pl.BlockSpec((RBLK, DP), lambda i: (i, 0)),
            pl.BlockSpec((DP, H1), full),
            pl.BlockSpec((1, H1), full),
            pl.BlockSpec((H2, H1), full),
            pl.BlockSpec((1, H2), full),
            pl.BlockSpec((16, H2), full),
            pl.BlockSpec((2, 16, 128), lambda i: (0, 0, 0)),
            pl.BlockSpec(memory_space=pltpu.SMEM),
            pl.BlockSpec((16, 1), full),
        ],
        out_specs=pl.BlockSpec((RBLK, 16, 128), lambda i: (i, 0, 0)),
        out_shape=jax.ShapeDtypeStruct((ROWS, 16, 128), jnp.float32),
    )(E, W1Tp, b1r, W2, b2r, W3X, pu16, wide_W, c16)


def kernel(product_id, user_id, year, month, day_of_week, hour,
           min_year, max_year,
           product_table, user_table, year_table, month_table,
           day_week_table, time_day_table,
           wide_W, wide_b, W1, b1, W2, b2, W3, b3):
    ids_all = jnp.concatenate(
        [product_id.T, user_id.T, year.T, month.T, day_of_week.T, hour.T],
        axis=0).reshape(-1).astype(jnp.int32)

    E = _sc_gather(ids_all, product_table.T, user_table.T,
                   year_table, month_table, day_week_table, time_day_table)

    W1Tp = jnp.zeros((DP, H1), jnp.float32).at[:D, :].set(W1.T)
    b1r = b1.reshape(1, H1)
    b2r = b2.reshape(1, H2)
    W3X = jnp.tile(W3, (8, 1))

    pu16 = jnp.repeat(
        jnp.concatenate([product_id.T, user_id.T], axis=0)
        .astype(jnp.float32).reshape(2, 8, 128), 2, axis=1)
    c16 = (wide_b[0] + jnp.tile(b3, 8)).reshape(16, 1)

    out3 = _mlp_call(E, W1Tp, b1r, W2, b2r, W3X, pu16, wide_W, c16)
    out4 = out3.reshape(ROWS, 8, 2, 128).transpose(0, 1, 3, 2)
    return out4.reshape(ROWS, B, 2)
